# baseline (speedup 1.0000x reference)
import functools

import jax
import jax.numpy as jnp
from jax import lax
from jax.experimental import pallas as pl
from jax.experimental.pallas import tpu as pltpu
from jax.experimental.pallas import tpu_sc as plsc

VOCAB = 1_000_000
D = 16
B = 16384
NC, NS, L = 2, 16, 16
NW = NC * NS
BPW = B // NW
CH = 128
NCH = BPW // CH
PAD = D + 1


def _sc_body(center_hbm, context_hbm, table_hbm, out_hbm,
             idx_c, idx_x, rows_c, rows_x, prod, out_v, sem):
    wid = lax.axis_index("s") * NC + lax.axis_index("c")
    base = wid * BPW

    pltpu.sync_copy(center_hbm.at[pl.ds(base, BPW)], idx_c)
    pltpu.sync_copy(context_hbm.at[pl.ds(base, BPW)], idx_x)

    copies = []
    for j in range(NCH):
        copies.append(pltpu.async_copy(
            table_hbm.at[idx_c.at[pl.ds(j * CH, CH)]],
            rows_c.at[pl.ds(j * CH, CH)], sem))
        copies.append(pltpu.async_copy(
            table_hbm.at[idx_x.at[pl.ds(j * CH, CH)]],
            rows_x.at[pl.ds(j * CH, CH)], sem))
    for c in copies:
        c.wait()

    lanes = lax.iota(jnp.int32, L)

    def group(g, carry):
        row0 = g * L
        for r in range(L):
            i = row0 + r
            prod[pl.ds(i * PAD, D)] = rows_c[i] * rows_x[i]
        acc = jnp.zeros((L,), jnp.float32)
        rowoff = (row0 + lanes) * PAD
        for d in range(D):
            acc = acc + plsc.load_gather(prod, [rowoff + d])
        out_v[pl.ds(row0, L)] = acc
        return carry

    lax.fori_loop(0, BPW // L, group, 0)
    pltpu.sync_copy(out_v, out_hbm.at[pl.ds(base, BPW)])


@jax.jit
def kernel(center, context, embedding_weight):
    mesh = plsc.VectorSubcoreMesh(
        core_axis_name="c", subcore_axis_name="s",
        num_cores=NC, num_subcores=NS)
    run = pl.kernel(
        _sc_body,
        out_type=jax.ShapeDtypeStruct((B,), jnp.float32),
        mesh=mesh,
        compiler_params=pltpu.CompilerParams(needs_layout_passes=False,
                                             use_tc_tiling_on_sc=False),
        scratch_types=[
            pltpu.VMEM((BPW,), jnp.int32),
            pltpu.VMEM((BPW,), jnp.int32),
            pltpu.VMEM((BPW, D), jnp.float32),
            pltpu.VMEM((BPW, D), jnp.float32),
            pltpu.VMEM((BPW * PAD,), jnp.float32),
            pltpu.VMEM((BPW,), jnp.float32),
            pltpu.SemaphoreType.DMA,
        ],
    )
    return run(center.astype(jnp.int32), context.astype(jnp.int32),
               embedding_weight)


# speedup vs baseline: 1.6374x; 1.6374x over previous
import jax
import jax.numpy as jnp
from jax import lax
from jax.experimental import pallas as pl
from jax.experimental.pallas import tpu as pltpu
from jax.experimental.pallas import tpu_sc as plsc

VOCAB = 1_000_000
D = 16
B = 16384
NC, NS, L = 2, 16, 16
NW = NC * NS
BPW = B // NW
CH = 128
NCH = BPW // CH
PAD = D + 1

VC = 32768
Q = VC // 8
TGRID = -(-VOCAB // VC)
PROWS = TGRID * Q


def _tc_pack_body(in_ref, out_ref):
    tt = in_ref[...].T
    out_ref[...] = jnp.concatenate(
        [tt[k * Q:(k + 1) * Q, :] for k in range(8)], axis=1)


def _tc_pack(table_t):
    return pl.pallas_call(
        _tc_pack_body,
        grid=(TGRID,),
        in_specs=[pl.BlockSpec((D, VC), lambda g: (0, g))],
        out_specs=pl.BlockSpec((Q, 128), lambda g: (g, 0)),
        out_shape=jax.ShapeDtypeStruct((PROWS, 128), jnp.float32),
    )(table_t)


def _remap(r):
    return (((jnp.right_shift(r, 15) << 12) | (r & 4095)) << 3) | (
        jnp.right_shift(r, 12) & 7)


def _sc_body(center_hbm, context_hbm, table_hbm, out_hbm,
             idx_c, idx_x, rows_c, rows_x, prod, out_v, sem):
    wid = lax.axis_index("s") * NC + lax.axis_index("c")
    base = wid * BPW

    pltpu.sync_copy(center_hbm.at[pl.ds(base, BPW)], idx_c)
    pltpu.sync_copy(context_hbm.at[pl.ds(base, BPW)], idx_x)

    def remap_chunk(t, carry):
        sl = pl.ds(t * L, L)
        idx_c[sl] = _remap(idx_c[sl])
        idx_x[sl] = _remap(idx_x[sl])
        return carry

    lax.fori_loop(0, BPW // L, remap_chunk, 0)

    copies = []
    for j in range(NCH):
        copies.append(pltpu.async_copy(
            table_hbm.at[idx_c.at[pl.ds(j * CH, CH)]],
            rows_c.at[pl.ds(j * CH, CH)], sem))
        copies.append(pltpu.async_copy(
            table_hbm.at[idx_x.at[pl.ds(j * CH, CH)]],
            rows_x.at[pl.ds(j * CH, CH)], sem))
    for c in copies:
        c.wait()

    lanes = lax.iota(jnp.int32, L)

    def group(g, carry):
        row0 = g * L
        for r in range(L):
            i = row0 + r
            prod[pl.ds(i * PAD, D)] = rows_c[i] * rows_x[i]
        acc = jnp.zeros((L,), jnp.float32)
        rowoff = (row0 + lanes) * PAD
        for d in range(D):
            acc = acc + plsc.load_gather(prod, [rowoff + d])
        out_v[pl.ds(row0, L)] = acc
        return carry

    lax.fori_loop(0, BPW // L, group, 0)
    pltpu.sync_copy(out_v, out_hbm.at[pl.ds(base, BPW)])


@jax.jit
def kernel(center, context, embedding_weight):
    mesh = plsc.VectorSubcoreMesh(
        core_axis_name="c", subcore_axis_name="s",
        num_cores=NC, num_subcores=NS)
    run = pl.kernel(
        _sc_body,
        out_type=jax.ShapeDtypeStruct((B,), jnp.float32),
        mesh=mesh,
        compiler_params=pltpu.CompilerParams(needs_layout_passes=False,
                                             use_tc_tiling_on_sc=False),
        scratch_types=[
            pltpu.VMEM((BPW,), jnp.int32),
            pltpu.VMEM((BPW,), jnp.int32),
            pltpu.VMEM((BPW, D), jnp.float32),
            pltpu.VMEM((BPW, D), jnp.float32),
            pltpu.VMEM((BPW * PAD,), jnp.float32),
            pltpu.VMEM((BPW,), jnp.float32),
            pltpu.SemaphoreType.DMA,
        ],
    )
    packed = _tc_pack(embedding_weight.T).reshape(PROWS * 8, D)
    return run(center.astype(jnp.int32), context.astype(jnp.int32),
               packed)


# speedup vs baseline: 5.0251x; 3.0690x over previous
import jax
import jax.numpy as jnp
from jax import lax
from jax.experimental import pallas as pl
from jax.experimental.pallas import tpu as pltpu
from jax.experimental.pallas import tpu_sc as plsc

VOCAB = 1_000_000
D = 16
B = 16384
NC, NS, L = 2, 16, 16
NW = NC * NS
BPW = B // NW
CH = 128
NCH = BPW // CH
PAD = D + 1

VC = 32768
Q = VC // 8
TGRID = -(-VOCAB // VC)
PROWS = TGRID * Q


def _tc_pack_body(in_ref, out_ref):
    x = in_ref[...]
    y = jnp.concatenate(
        [x[:, k * Q:(k + 1) * Q] for k in range(8)], axis=0)
    ident = jnp.eye(128, dtype=jnp.float32)
    out_ref[...] = lax.dot_general(
        y, ident, (((0,), (0,)), ((), ())),
        precision=lax.Precision.HIGHEST,
        preferred_element_type=jnp.float32)


def _tc_pack(table_t):
    return pl.pallas_call(
        _tc_pack_body,
        grid=(TGRID,),
        in_specs=[pl.BlockSpec((D, VC), lambda g: (0, g))],
        out_specs=pl.BlockSpec((Q, 128), lambda g: (g, 0)),
        out_shape=jax.ShapeDtypeStruct((PROWS, 128), jnp.float32),
    )(table_t)


def _remap(r):
    return (((jnp.right_shift(r, 15) << 12) | (r & 4095)) << 3) | (
        jnp.right_shift(r, 12) & 7)


def _sc_body(center_hbm, context_hbm, table_hbm, out_hbm,
             idx_c, idx_x, rows_c, rows_x, prod, out_v, sem):
    wid = lax.axis_index("s") * NC + lax.axis_index("c")
    base = wid * BPW

    pltpu.sync_copy(center_hbm.at[pl.ds(base, BPW)], idx_c)
    pltpu.sync_copy(context_hbm.at[pl.ds(base, BPW)], idx_x)

    def remap_chunk(t, carry):
        sl = pl.ds(t * L, L)
        idx_c[sl] = _remap(idx_c[sl])
        idx_x[sl] = _remap(idx_x[sl])
        return carry

    lax.fori_loop(0, BPW // L, remap_chunk, 0)

    copies = []
    for j in range(NCH):
        copies.append(pltpu.async_copy(
            table_hbm.at[idx_c.at[pl.ds(j * CH, CH)]],
            rows_c.at[pl.ds(j * CH, CH)], sem))
        copies.append(pltpu.async_copy(
            table_hbm.at[idx_x.at[pl.ds(j * CH, CH)]],
            rows_x.at[pl.ds(j * CH, CH)], sem))
    for c in copies:
        c.wait()

    lanes = lax.iota(jnp.int32, L)

    def group(g, carry):
        row0 = g * L
        for r in range(L):
            i = row0 + r
            prod[pl.ds(i * PAD, D)] = rows_c[i] * rows_x[i]
        acc = jnp.zeros((L,), jnp.float32)
        rowoff = (row0 + lanes) * PAD
        for d in range(D):
            acc = acc + plsc.load_gather(prod, [rowoff + d])
        out_v[pl.ds(row0, L)] = acc
        return carry

    lax.fori_loop(0, BPW // L, group, 0)
    pltpu.sync_copy(out_v, out_hbm.at[pl.ds(base, BPW)])


@jax.jit
def kernel(center, context, embedding_weight):
    mesh = plsc.VectorSubcoreMesh(
        core_axis_name="c", subcore_axis_name="s",
        num_cores=NC, num_subcores=NS)
    run = pl.kernel(
        _sc_body,
        out_type=jax.ShapeDtypeStruct((B,), jnp.float32),
        mesh=mesh,
        compiler_params=pltpu.CompilerParams(needs_layout_passes=False,
                                             use_tc_tiling_on_sc=False),
        scratch_types=[
            pltpu.VMEM((BPW,), jnp.int32),
            pltpu.VMEM((BPW,), jnp.int32),
            pltpu.VMEM((BPW, D), jnp.float32),
            pltpu.VMEM((BPW, D), jnp.float32),
            pltpu.VMEM((BPW * PAD,), jnp.float32),
            pltpu.VMEM((BPW,), jnp.float32),
            pltpu.SemaphoreType.DMA,
        ],
    )
    packed = _tc_pack(embedding_weight.T).reshape(PROWS * 8, D)
    return run(center.astype(jnp.int32), context.astype(jnp.int32),
               packed)


# speedup vs baseline: 6.3625x; 1.2661x over previous
import jax
import jax.numpy as jnp
from jax import lax
from jax.experimental import pallas as pl
from jax.experimental.pallas import tpu as pltpu
from jax.experimental.pallas import tpu_sc as plsc

VOCAB = 1_000_000
D = 16
B = 16384
NC, NS, L = 2, 16, 16
NW = NC * NS
BPW = B // NW
CH = 128
NCH = BPW // CH
PAD = D + 1

VC = 32768
Q = VC // 8
TGRID = -(-VOCAB // VC)
PROWS = TGRID * Q


def _tc_pack_body(in_ref, out_ref):
    x = in_ref[...]
    y = jnp.concatenate(
        [x[:, k * Q:(k + 1) * Q] for k in range(8)], axis=0)
    out_ref[...] = y.T


def _tc_pack(table_t):
    return pl.pallas_call(
        _tc_pack_body,
        grid=(TGRID,),
        in_specs=[pl.BlockSpec((D, VC), lambda g: (0, g))],
        out_specs=pl.BlockSpec((Q, 128), lambda g: (g, 0)),
        out_shape=jax.ShapeDtypeStruct((PROWS, 128), jnp.float32),
    )(table_t)


def _remap(r):
    return (((jnp.right_shift(r, 15) << 12) | (r & 4095)) << 3) | (
        jnp.right_shift(r, 12) & 7)


def _sc_body(center_hbm, context_hbm, table_hbm, out_hbm,
             idx_c, idx_x, rows_c, rows_x, prod, out_v, sem):
    wid = lax.axis_index("s") * NC + lax.axis_index("c")
    base = wid * BPW

    pltpu.sync_copy(center_hbm.at[pl.ds(base, BPW)], idx_c)
    pltpu.sync_copy(context_hbm.at[pl.ds(base, BPW)], idx_x)

    def remap_chunk(t, carry):
        sl = pl.ds(t * L, L)
        idx_c[sl] = _remap(idx_c[sl])
        idx_x[sl] = _remap(idx_x[sl])
        return carry

    lax.fori_loop(0, BPW // L, remap_chunk, 0)

    copies = []
    for j in range(NCH):
        copies.append(pltpu.async_copy(
            table_hbm.at[idx_c.at[pl.ds(j * CH, CH)]],
            rows_c.at[pl.ds(j * CH, CH)], sem))
        copies.append(pltpu.async_copy(
            table_hbm.at[idx_x.at[pl.ds(j * CH, CH)]],
            rows_x.at[pl.ds(j * CH, CH)], sem))
    for c in copies:
        c.wait()

    lanes = lax.iota(jnp.int32, L)

    def group(g, carry):
        row0 = g * L
        for r in range(L):
            i = row0 + r
            prod[pl.ds(i * PAD, D)] = rows_c[i] * rows_x[i]
        acc = jnp.zeros((L,), jnp.float32)
        rowoff = (row0 + lanes) * PAD
        for d in range(D):
            acc = acc + plsc.load_gather(prod, [rowoff + d])
        out_v[pl.ds(row0, L)] = acc
        return carry

    lax.fori_loop(0, BPW // L, group, 0)
    pltpu.sync_copy(out_v, out_hbm.at[pl.ds(base, BPW)])


@jax.jit
def kernel(center, context, embedding_weight):
    mesh = plsc.VectorSubcoreMesh(
        core_axis_name="c", subcore_axis_name="s",
        num_cores=NC, num_subcores=NS)
    run = pl.kernel(
        _sc_body,
        out_type=jax.ShapeDtypeStruct((B,), jnp.float32),
        mesh=mesh,
        compiler_params=pltpu.CompilerParams(needs_layout_passes=False,
                                             use_tc_tiling_on_sc=False),
        scratch_types=[
            pltpu.VMEM((BPW,), jnp.int32),
            pltpu.VMEM((BPW,), jnp.int32),
            pltpu.VMEM((BPW, D), jnp.float32),
            pltpu.VMEM((BPW, D), jnp.float32),
            pltpu.VMEM((BPW * PAD,), jnp.float32),
            pltpu.VMEM((BPW,), jnp.float32),
            pltpu.SemaphoreType.DMA,
        ],
    )
    packed = _tc_pack(embedding_weight.T).reshape(PROWS * 8, D)
    return run(center.astype(jnp.int32), context.astype(jnp.int32),
               packed)
